# Initial kernel scaffold; baseline (speedup 1.0000x reference)
import jax
import jax.numpy as jnp
from jax.experimental import pallas as pl


def kernel(x, params, edge_index):
    raise NotImplementedError("write your pallas kernel here")


# speedup vs baseline: 6.1677x; 6.1677x over previous
import functools

import jax
import jax.numpy as jnp
from jax import lax
from jax.experimental import pallas as pl
from jax.experimental.pallas import tpu as pltpu
from jax.experimental.pallas import tpu_sc as plsc

NC, NS = 2, 16
NW = NC * NS
CH = 128
ZB = 128
EPS = 1e-5


def _pad_n(n):
    q = NS * ZB
    return (n + q - 1) // q * q


def _sc_mesh():
    return plsc.VectorSubcoreMesh(core_axis_name="c", subcore_axis_name="s")


@functools.lru_cache(maxsize=None)
def _sc_degrees(E, N):
    nchunk = E // CH
    NP = _pad_n(N)
    rows_per = NP // NS

    @functools.partial(
        pl.kernel,
        out_type=(jax.ShapeDtypeStruct((NC, NP, 16), jnp.float32),
                  jax.ShapeDtypeStruct((NC, NP, 16), jnp.float32)),
        mesh=_sc_mesh(),
        scratch_types=(
            pltpu.VMEM((CH,), jnp.int32),
            pltpu.VMEM((CH,), jnp.int32),
            pltpu.VMEM((CH, 16), jnp.float32),
            pltpu.VMEM((ZB, 16), jnp.float32),
            pltpu.VMEM_SHARED((NP, 16), jnp.float32),
            pltpu.VMEM_SHARED((NP, 16), jnp.float32),
        ),
        compiler_params=pltpu.CompilerParams(use_tc_tiling_on_sc=False),
    )
    def deg_kernel(src_hbm, dst_hbm, ones_hbm, zeros_hbm, outs, outd,
                   sidx, didx, ones_v, buf, acc_s, acc_d):
        c = lax.axis_index("c")
        s = lax.axis_index("s")
        w = s * NC + c
        base = s * rows_per
        pltpu.sync_copy(zeros_hbm, buf)
        for k in range(rows_per // ZB):
            pltpu.sync_copy(buf, acc_s.at[pl.ds(base + k * ZB, ZB)])
            pltpu.sync_copy(buf, acc_d.at[pl.ds(base + k * ZB, ZB)])
        pltpu.sync_copy(ones_hbm, ones_v)
        plsc.subcore_barrier()

        cnt = (nchunk - w + NW - 1) // NW

        def body(g, carry):
            off = (w + g * NW) * CH
            pltpu.sync_copy(src_hbm.at[pl.ds(off, CH)], sidx)
            pltpu.sync_copy(dst_hbm.at[pl.ds(off, CH)], didx)
            pltpu.sync_copy(ones_v, acc_s.at[sidx], add=True)
            pltpu.sync_copy(ones_v, acc_d.at[didx], add=True)
            return carry

        lax.fori_loop(0, cnt, body, 0)
        plsc.subcore_barrier()
        for k in range(rows_per // ZB):
            pltpu.sync_copy(acc_s.at[pl.ds(base + k * ZB, ZB)], buf)
            pltpu.sync_copy(buf, outs.at[c, pl.ds(base + k * ZB, ZB)])
        for k in range(rows_per // ZB):
            pltpu.sync_copy(acc_d.at[pl.ds(base + k * ZB, ZB)], buf)
            pltpu.sync_copy(buf, outd.at[c, pl.ds(base + k * ZB, ZB)])

    return deg_kernel


@functools.lru_cache(maxsize=None)
def _sc_segsum(E, N, D):
    nchunk = E // CH
    NP = _pad_n(N)
    rows_per = NP // NS

    @functools.partial(
        pl.kernel,
        out_type=jax.ShapeDtypeStruct((NC, NP, D), jnp.float32),
        mesh=_sc_mesh(),
        scratch_types=(
            pltpu.VMEM((CH,), jnp.int32),
            pltpu.VMEM((CH,), jnp.int32),
            pltpu.VMEM((CH, D), jnp.float32),
            pltpu.VMEM((ZB, D), jnp.float32),
            pltpu.VMEM_SHARED((NP, D), jnp.float32),
            pltpu.SemaphoreType.DMA,
        ),
        compiler_params=pltpu.CompilerParams(use_tc_tiling_on_sc=False),
    )
    def seg_kernel(h_hbm, src_hbm, dst_hbm, zeros_hbm, out,
                   sidx, didx, rows, buf, acc, sem):
        c = lax.axis_index("c")
        s = lax.axis_index("s")
        w = s * NC + c
        base = s * rows_per
        pltpu.sync_copy(zeros_hbm, buf)
        for k in range(rows_per // ZB):
            pltpu.sync_copy(buf, acc.at[pl.ds(base + k * ZB, ZB)])
        plsc.subcore_barrier()

        cnt = (nchunk - w + NW - 1) // NW

        def body(g, carry):
            off = (w + g * NW) * CH
            pltpu.sync_copy(src_hbm.at[pl.ds(off, CH)], sidx)
            pltpu.sync_copy(dst_hbm.at[pl.ds(off, CH)], didx)
            pltpu.async_copy(h_hbm.at[sidx], rows, sem).wait()
            pltpu.sync_copy(rows, acc.at[didx], add=True)
            return carry

        lax.fori_loop(0, cnt, body, 0)
        plsc.subcore_barrier()
        for k in range(rows_per // ZB):
            pltpu.sync_copy(acc.at[pl.ds(base + k * ZB, ZB)], buf)
            pltpu.sync_copy(buf, out.at[c, pl.ds(base + k * ZB, ZB)])

    return seg_kernel


def _tc0_body(n, x_ref, ds_ref, dd_ref, w_ref, hpre_ref, norms_ref):
    d_s = ds_ref[...]
    d_d = dd_ref[...]
    deg_o = d_s[0, :n, 0:1] + d_s[1, :n, 0:1]
    deg_i = d_d[0, :n, 0:1] + d_d[1, :n, 0:1]
    ns = jnp.where(deg_o > 0, lax.rsqrt(jnp.maximum(deg_o, 1.0)), 0.0)
    nd = jnp.where(deg_i > 0, lax.rsqrt(jnp.maximum(deg_i, 1.0)), 0.0)
    norms_ref[...] = jnp.concatenate([ns, nd], axis=1)
    hpre_ref[...] = jnp.dot(x_ref[...] * ns, w_ref[...],
                            preferred_element_type=jnp.float32)


def _post_agg(n, parts_ref, norms_ref, b_ref, g_ref, beta_ref, a_ref):
    p = parts_ref[...]
    agg = p[0, :n] + p[1, :n]
    nrm = norms_ref[...]
    ns = nrm[:, 0:1]
    nd = nrm[:, 1:2]
    h = agg * nd + b_ref[...]
    mu = jnp.mean(h, axis=0, keepdims=True)
    var = jnp.mean((h - mu) ** 2, axis=0, keepdims=True)
    hn = (h - mu) * lax.rsqrt(var + EPS) * g_ref[...] + beta_ref[...]
    hp = jnp.where(hn >= 0, hn, a_ref[...] * hn)
    return hp, ns


def _tc_mid_body(n, parts_ref, norms_ref, b_ref, g_ref, beta_ref, a_ref,
                 w_ref, out_ref):
    hp, ns = _post_agg(n, parts_ref, norms_ref, b_ref, g_ref, beta_ref, a_ref)
    out_ref[...] = jnp.dot(hp * ns, w_ref[...],
                           preferred_element_type=jnp.float32)


def _tc_final_body(n, parts_ref, norms_ref, b_ref, g_ref, beta_ref, a_ref,
                   wc_ref, bc_ref, out_ref):
    hp, _ = _post_agg(n, parts_ref, norms_ref, b_ref, g_ref, beta_ref, a_ref)
    hg = jnp.mean(hp, axis=0, keepdims=True)
    out_ref[...] = jnp.dot(hg, wc_ref[...],
                           preferred_element_type=jnp.float32) + bc_ref[...]


def _tc(body, out_shape, *args):
    return pl.pallas_call(body, out_shape=out_shape)(*args)


def kernel(x, params, edge_index):
    N, F = x.shape
    E = edge_index.shape[1]
    f32 = jnp.float32
    src = edge_index[0]
    dst = edge_index[1]

    dims = [F] + [params[f"W{i}"].shape[1] for i in range(6)]
    dp = [max(d, 16) for d in dims]

    ws, bs, gs, betas = [], [], [], []
    for i in range(6):
        wi = params[f"W{i}"].astype(f32)
        ws.append(jnp.pad(wi, ((0, dp[i] - wi.shape[0]),
                               (0, dp[i + 1] - wi.shape[1]))))
        padc = dp[i + 1] - dims[i + 1]
        bs.append(jnp.pad(params[f"b{i}"], (0, padc))[None, :])
        gs.append(jnp.pad(params[f"g{i}"], (0, padc), constant_values=1.0)[None, :])
        betas.append(jnp.pad(params[f"beta{i}"], (0, padc))[None, :])
    a = params["a"].reshape(1, 1)
    wc = jnp.pad(params["Wc"], ((0, dp[6] - params["Wc"].shape[0]), (0, 0)))
    bc = params["bc"][None, :]

    ones16 = jnp.ones((CH, 16), f32)
    zeros = {d: jnp.zeros((ZB, d), f32) for d in sorted(set(dp[1:]) | {16})}

    degs_s, degs_d = _sc_degrees(E, N)(src, dst, ones16, zeros[16])
    hpre, norms = _tc(
        functools.partial(_tc0_body, N),
        (jax.ShapeDtypeStruct((N, dp[1]), f32),
         jax.ShapeDtypeStruct((N, 2), f32)),
        x, degs_s, degs_d, ws[0])
    for i in range(6):
        d = dp[i + 1]
        parts = _sc_segsum(E, N, d)(hpre, src, dst, zeros[d])
        if i < 5:
            hpre = _tc(
                functools.partial(_tc_mid_body, N),
                jax.ShapeDtypeStruct((N, dp[i + 2]), f32),
                parts, norms, bs[i], gs[i], betas[i], a, ws[i + 1])
        else:
            out = _tc(
                functools.partial(_tc_final_body, N),
                jax.ShapeDtypeStruct((1, bc.shape[1]), f32),
                parts, norms, bs[i], gs[i], betas[i], a, wc, bc)
    return out


# speedup vs baseline: 12.7466x; 2.0667x over previous
import functools

import jax
import jax.numpy as jnp
from jax import lax
from jax.experimental import pallas as pl
from jax.experimental.pallas import tpu as pltpu
from jax.experimental.pallas import tpu_sc as plsc

NC, NS = 2, 16
NW = NC * NS
CH = 128
ZB = 128
EPS = 1e-5


def _pad_n(n):
    q = NS * ZB
    return (n + q - 1) // q * q


def _sc_mesh():
    return plsc.VectorSubcoreMesh(core_axis_name="c", subcore_axis_name="s")


@functools.lru_cache(maxsize=None)
def _sc_degrees(E, N):
    nchunk = E // CH
    K = 10
    nblk = nchunk // K
    assert nchunk % K == 0
    NP = _pad_n(N)
    rows_per = NP // NS

    @functools.partial(
        pl.kernel,
        out_type=(jax.ShapeDtypeStruct((NC, NP, 16), jnp.float32),
                  jax.ShapeDtypeStruct((NC, NP, 16), jnp.float32)),
        mesh=_sc_mesh(),
        scratch_types=(
            pltpu.VMEM((K, CH), jnp.int32),
            pltpu.VMEM((K, CH), jnp.int32),
            pltpu.VMEM((CH, 16), jnp.float32),
            pltpu.VMEM((ZB, 16), jnp.float32),
            pltpu.VMEM_SHARED((NP, 16), jnp.float32),
            pltpu.VMEM_SHARED((NP, 16), jnp.float32),
        ),
        compiler_params=pltpu.CompilerParams(use_tc_tiling_on_sc=False),
    )
    def deg_kernel(src_hbm, dst_hbm, ones_hbm, zeros_hbm, outs, outd,
                   sidx, didx, ones_v, buf, acc_s, acc_d):
        c = lax.axis_index("c")
        s = lax.axis_index("s")
        w = s * NC + c
        base = s * rows_per
        pltpu.sync_copy(zeros_hbm, buf)
        for k in range(rows_per // ZB):
            pltpu.sync_copy(buf, acc_s.at[pl.ds(base + k * ZB, ZB)])
            pltpu.sync_copy(buf, acc_d.at[pl.ds(base + k * ZB, ZB)])
        pltpu.sync_copy(ones_hbm, ones_v)
        plsc.subcore_barrier()

        cnt = (nblk - w + NW - 1) // NW

        def body(g, carry):
            blk = w + g * NW
            pltpu.sync_copy(src_hbm.at[pl.ds(blk * K, K)], sidx)
            pltpu.sync_copy(dst_hbm.at[pl.ds(blk * K, K)], didx)
            for j in range(K):
                pltpu.sync_copy(ones_v, acc_s.at[sidx.at[j]], add=True)
                pltpu.sync_copy(ones_v, acc_d.at[didx.at[j]], add=True)
            return carry

        lax.fori_loop(0, cnt, body, 0)
        plsc.subcore_barrier()
        for k in range(rows_per // ZB):
            pltpu.sync_copy(acc_s.at[pl.ds(base + k * ZB, ZB)], buf)
            pltpu.sync_copy(buf, outs.at[c, pl.ds(base + k * ZB, ZB)])
        for k in range(rows_per // ZB):
            pltpu.sync_copy(acc_d.at[pl.ds(base + k * ZB, ZB)], buf)
            pltpu.sync_copy(buf, outd.at[c, pl.ds(base + k * ZB, ZB)])

    return deg_kernel


@functools.lru_cache(maxsize=None)
def _sc_segsum(E, N, D):
    nchunk = E // CH
    K = 2 if D > 64 else (5 if D > 32 else 10)
    ZBl = 64 if D > 64 else ZB
    nblk = nchunk // K
    assert nchunk % K == 0
    NP = _pad_n(N)
    rows_per = NP // NS

    @functools.partial(
        pl.kernel,
        out_type=jax.ShapeDtypeStruct((NC, NP, D), jnp.float32),
        mesh=_sc_mesh(),
        scratch_types=(
            pltpu.VMEM((K, CH), jnp.int32),
            pltpu.VMEM((K, CH), jnp.int32),
            pltpu.VMEM((K, CH, D), jnp.float32),
            pltpu.VMEM((ZBl, D), jnp.float32),
            pltpu.VMEM_SHARED((NP, D), jnp.float32),
            pltpu.SemaphoreType.DMA,
        ),
        compiler_params=pltpu.CompilerParams(use_tc_tiling_on_sc=False),
    )
    def seg_kernel(h_hbm, src_hbm, dst_hbm, zeros_hbm, out,
                   sidx, didx, rows, buf, acc, sem):
        c = lax.axis_index("c")
        s = lax.axis_index("s")
        w = s * NC + c
        base = s * rows_per
        pltpu.sync_copy(zeros_hbm, buf)
        for k in range(rows_per // ZBl):
            pltpu.sync_copy(buf, acc.at[pl.ds(base + k * ZBl, ZBl)])
        plsc.subcore_barrier()

        cnt = (nblk - w + NW - 1) // NW

        def body(g, carry):
            blk = w + g * NW
            pltpu.sync_copy(src_hbm.at[pl.ds(blk * K, K)], sidx)
            pltpu.sync_copy(dst_hbm.at[pl.ds(blk * K, K)], didx)
            handles = [pltpu.async_copy(h_hbm.at[sidx.at[j]], rows.at[j], sem)
                       for j in range(K)]
            for j in range(K):
                handles[j].wait()
                pltpu.sync_copy(rows.at[j], acc.at[didx.at[j]], add=True)
            return carry

        lax.fori_loop(0, cnt, body, 0)
        plsc.subcore_barrier()
        for k in range(rows_per // ZBl):
            pltpu.sync_copy(acc.at[pl.ds(base + k * ZBl, ZBl)], buf)
            pltpu.sync_copy(buf, out.at[c, pl.ds(base + k * ZBl, ZBl)])

    return seg_kernel


def _tc0_body(n, x_ref, ds_ref, dd_ref, w_ref, hpre_ref, norms_ref):
    d_s = ds_ref[...]
    d_d = dd_ref[...]
    deg_o = d_s[0, :n, 0:1] + d_s[1, :n, 0:1]
    deg_i = d_d[0, :n, 0:1] + d_d[1, :n, 0:1]
    ns = jnp.where(deg_o > 0, lax.rsqrt(jnp.maximum(deg_o, 1.0)), 0.0)
    nd = jnp.where(deg_i > 0, lax.rsqrt(jnp.maximum(deg_i, 1.0)), 0.0)
    norms_ref[...] = jnp.concatenate([ns, nd], axis=1)
    hpre_ref[...] = jnp.dot(x_ref[...] * ns, w_ref[...],
                            preferred_element_type=jnp.float32)


def _post_agg(n, parts_ref, norms_ref, b_ref, g_ref, beta_ref, a_ref):
    p = parts_ref[...]
    agg = p[0, :n] + p[1, :n]
    nrm = norms_ref[...]
    ns = nrm[:, 0:1]
    nd = nrm[:, 1:2]
    h = agg * nd + b_ref[...]
    mu = jnp.mean(h, axis=0, keepdims=True)
    var = jnp.mean((h - mu) ** 2, axis=0, keepdims=True)
    hn = (h - mu) * lax.rsqrt(var + EPS) * g_ref[...] + beta_ref[...]
    hp = jnp.where(hn >= 0, hn, a_ref[...] * hn)
    return hp, ns


def _tc_mid_body(n, parts_ref, norms_ref, b_ref, g_ref, beta_ref, a_ref,
                 w_ref, out_ref):
    hp, ns = _post_agg(n, parts_ref, norms_ref, b_ref, g_ref, beta_ref, a_ref)
    out_ref[...] = jnp.dot(hp * ns, w_ref[...],
                           preferred_element_type=jnp.float32)


def _tc_final_body(n, parts_ref, norms_ref, b_ref, g_ref, beta_ref, a_ref,
                   wc_ref, bc_ref, out_ref):
    hp, _ = _post_agg(n, parts_ref, norms_ref, b_ref, g_ref, beta_ref, a_ref)
    hg = jnp.mean(hp, axis=0, keepdims=True)
    out_ref[...] = jnp.dot(hg, wc_ref[...],
                           preferred_element_type=jnp.float32) + bc_ref[...]


def _tc(body, out_shape, *args):
    return pl.pallas_call(body, out_shape=out_shape)(*args)


def kernel(x, params, edge_index):
    N, F = x.shape
    E = edge_index.shape[1]
    f32 = jnp.float32
    src = edge_index[0].reshape(E // CH, CH)
    dst = edge_index[1].reshape(E // CH, CH)

    dims = [F] + [params[f"W{i}"].shape[1] for i in range(6)]
    dp = [max(d, 16) for d in dims]

    ws, bs, gs, betas = [], [], [], []
    for i in range(6):
        wi = params[f"W{i}"].astype(f32)
        ws.append(jnp.pad(wi, ((0, dp[i] - wi.shape[0]),
                               (0, dp[i + 1] - wi.shape[1]))))
        padc = dp[i + 1] - dims[i + 1]
        bs.append(jnp.pad(params[f"b{i}"], (0, padc))[None, :])
        gs.append(jnp.pad(params[f"g{i}"], (0, padc), constant_values=1.0)[None, :])
        betas.append(jnp.pad(params[f"beta{i}"], (0, padc))[None, :])
    a = params["a"].reshape(1, 1)
    wc = jnp.pad(params["Wc"], ((0, dp[6] - params["Wc"].shape[0]), (0, 0)))
    bc = params["bc"][None, :]

    ones16 = jnp.ones((CH, 16), f32)
    zeros = {d: jnp.zeros((64 if d > 64 else ZB, d), f32)
             for d in sorted(set(dp[1:]))}
    zeros16 = jnp.zeros((ZB, 16), f32)

    degs_s, degs_d = _sc_degrees(E, N)(src, dst, ones16, zeros16)
    hpre, norms = _tc(
        functools.partial(_tc0_body, N),
        (jax.ShapeDtypeStruct((N, dp[1]), f32),
         jax.ShapeDtypeStruct((N, 2), f32)),
        x, degs_s, degs_d, ws[0])
    for i in range(6):
        d = dp[i + 1]
        parts = _sc_segsum(E, N, d)(hpre, src, dst, zeros[d])
        if i < 5:
            hpre = _tc(
                functools.partial(_tc_mid_body, N),
                jax.ShapeDtypeStruct((N, dp[i + 2]), f32),
                parts, norms, bs[i], gs[i], betas[i], a, ws[i + 1])
        else:
            out = _tc(
                functools.partial(_tc_final_body, N),
                jax.ShapeDtypeStruct((1, bc.shape[1]), f32),
                parts, norms, bs[i], gs[i], betas[i], a, wc, bc)
    return out


# speedup vs baseline: 14.5030x; 1.1378x over previous
import functools

import jax
import jax.numpy as jnp
from jax import lax
from jax.experimental import pallas as pl
from jax.experimental.pallas import tpu as pltpu
from jax.experimental.pallas import tpu_sc as plsc

NC, NS = 2, 16
NW = NC * NS
CH = 128
ZB = 128
EPS = 1e-5


def _pad_n(n):
    q = NS * ZB
    return (n + q - 1) // q * q


def _sc_mesh():
    return plsc.VectorSubcoreMesh(core_axis_name="c", subcore_axis_name="s")


@functools.lru_cache(maxsize=None)
def _sc_degrees(E, N):
    nchunk = E // CH
    K = 10
    nblk = nchunk // K
    assert nchunk % K == 0
    NP = _pad_n(N)
    rows_per = NP // NS

    @functools.partial(
        pl.kernel,
        out_type=(jax.ShapeDtypeStruct((NC, NP, 16), jnp.float32),
                  jax.ShapeDtypeStruct((NC, NP, 16), jnp.float32)),
        mesh=_sc_mesh(),
        scratch_types=(
            pltpu.VMEM((K, 2, CH), jnp.int32),
            pltpu.VMEM((CH, 16), jnp.float32),
            pltpu.VMEM((ZB, 16), jnp.float32),
            pltpu.VMEM_SHARED((NP, 16), jnp.float32),
            pltpu.VMEM_SHARED((NP, 16), jnp.float32),
            pltpu.SemaphoreType.DMA,
        ),
        compiler_params=pltpu.CompilerParams(use_tc_tiling_on_sc=False),
    )
    def deg_kernel(eidx_hbm, ones_hbm, zeros_hbm, outs, outd,
                   idxb, ones_v, buf, acc_s, acc_d, sem):
        c = lax.axis_index("c")
        s = lax.axis_index("s")
        w = s * NC + c
        base = s * rows_per
        pltpu.sync_copy(zeros_hbm, buf)
        for k in range(rows_per // ZB):
            pltpu.sync_copy(buf, acc_s.at[pl.ds(base + k * ZB, ZB)])
            pltpu.sync_copy(buf, acc_d.at[pl.ds(base + k * ZB, ZB)])
        pltpu.sync_copy(ones_hbm, ones_v)
        plsc.subcore_barrier()

        cnt = (nblk - w + NW - 1) // NW

        def body(g, carry):
            blk = w + g * NW
            pltpu.sync_copy(eidx_hbm.at[pl.ds(blk * K, K)], idxb)
            hs = []
            for j in range(K):
                hs.append(pltpu.async_copy(
                    ones_v, acc_s.at[idxb.at[j, 0]], sem, add=True))
                hs.append(pltpu.async_copy(
                    ones_v, acc_d.at[idxb.at[j, 1]], sem, add=True))
            for h in hs:
                h.wait()
            return carry

        lax.fori_loop(0, cnt, body, 0)
        plsc.subcore_barrier()
        for k in range(rows_per // ZB):
            pltpu.sync_copy(acc_s.at[pl.ds(base + k * ZB, ZB)], buf)
            pltpu.sync_copy(buf, outs.at[c, pl.ds(base + k * ZB, ZB)])
        for k in range(rows_per // ZB):
            pltpu.sync_copy(acc_d.at[pl.ds(base + k * ZB, ZB)], buf)
            pltpu.sync_copy(buf, outd.at[c, pl.ds(base + k * ZB, ZB)])

    return deg_kernel


@functools.lru_cache(maxsize=None)
def _sc_segsum(E, N, D):
    nchunk = E // CH
    K = 2 if D > 64 else (5 if D > 32 else 10)
    ZBl = 64 if D > 64 else ZB
    nblk = nchunk // K
    assert nchunk % K == 0
    NP = _pad_n(N)
    rows_per = NP // NS

    @functools.partial(
        pl.kernel,
        out_type=jax.ShapeDtypeStruct((NC, NP, D), jnp.float32),
        mesh=_sc_mesh(),
        scratch_types=(
            pltpu.VMEM((K, 2, CH), jnp.int32),
            pltpu.VMEM((K, CH, D), jnp.float32),
            pltpu.VMEM((ZBl, D), jnp.float32),
            pltpu.VMEM_SHARED((NP, D), jnp.float32),
            pltpu.SemaphoreType.DMA,
            pltpu.SemaphoreType.DMA,
        ),
        compiler_params=pltpu.CompilerParams(use_tc_tiling_on_sc=False),
    )
    def seg_kernel(h_hbm, eidx_hbm, zeros_hbm, out,
                   idxb, rows, buf, acc, gsem, ssem):
        c = lax.axis_index("c")
        s = lax.axis_index("s")
        w = s * NC + c
        base = s * rows_per
        pltpu.sync_copy(zeros_hbm, buf)
        for k in range(rows_per // ZBl):
            pltpu.sync_copy(buf, acc.at[pl.ds(base + k * ZBl, ZBl)])
        plsc.subcore_barrier()

        cnt = (nblk - w + NW - 1) // NW

        def body(g, carry):
            blk = w + g * NW
            pltpu.sync_copy(eidx_hbm.at[pl.ds(blk * K, K)], idxb)
            gh = [pltpu.async_copy(h_hbm.at[idxb.at[j, 0]], rows.at[j], gsem)
                  for j in range(K)]
            sh = []
            for j in range(K):
                gh[j].wait()
                sh.append(pltpu.async_copy(
                    rows.at[j], acc.at[idxb.at[j, 1]], ssem, add=True))
            for h in sh:
                h.wait()
            return carry

        lax.fori_loop(0, cnt, body, 0)
        plsc.subcore_barrier()
        for k in range(rows_per // ZBl):
            pltpu.sync_copy(acc.at[pl.ds(base + k * ZBl, ZBl)], buf)
            pltpu.sync_copy(buf, out.at[c, pl.ds(base + k * ZBl, ZBl)])

    return seg_kernel


def _tc0_body(n, x_ref, ds_ref, dd_ref, w_ref, hpre_ref, norms_ref):
    d_s = ds_ref[...]
    d_d = dd_ref[...]
    deg_o = d_s[0, :n, 0:1] + d_s[1, :n, 0:1]
    deg_i = d_d[0, :n, 0:1] + d_d[1, :n, 0:1]
    ns = jnp.where(deg_o > 0, lax.rsqrt(jnp.maximum(deg_o, 1.0)), 0.0)
    nd = jnp.where(deg_i > 0, lax.rsqrt(jnp.maximum(deg_i, 1.0)), 0.0)
    norms_ref[...] = jnp.concatenate([ns, nd], axis=1)
    hpre_ref[...] = jnp.dot(x_ref[...] * ns, w_ref[...],
                            preferred_element_type=jnp.float32)


def _post_agg(n, parts_ref, norms_ref, b_ref, g_ref, beta_ref, a_ref):
    p = parts_ref[...]
    agg = p[0, :n] + p[1, :n]
    nrm = norms_ref[...]
    ns = nrm[:, 0:1]
    nd = nrm[:, 1:2]
    h = agg * nd + b_ref[...]
    mu = jnp.mean(h, axis=0, keepdims=True)
    var = jnp.mean((h - mu) ** 2, axis=0, keepdims=True)
    hn = (h - mu) * lax.rsqrt(var + EPS) * g_ref[...] + beta_ref[...]
    hp = jnp.where(hn >= 0, hn, a_ref[...] * hn)
    return hp, ns


def _tc_mid_body(n, parts_ref, norms_ref, b_ref, g_ref, beta_ref, a_ref,
                 w_ref, out_ref):
    hp, ns = _post_agg(n, parts_ref, norms_ref, b_ref, g_ref, beta_ref, a_ref)
    out_ref[...] = jnp.dot(hp * ns, w_ref[...],
                           preferred_element_type=jnp.float32)


def _tc_final_body(n, parts_ref, norms_ref, b_ref, g_ref, beta_ref, a_ref,
                   wc_ref, bc_ref, out_ref):
    hp, _ = _post_agg(n, parts_ref, norms_ref, b_ref, g_ref, beta_ref, a_ref)
    hg = jnp.mean(hp, axis=0, keepdims=True)
    out_ref[...] = jnp.dot(hg, wc_ref[...],
                           preferred_element_type=jnp.float32) + bc_ref[...]


def _tc(body, out_shape, *args):
    return pl.pallas_call(body, out_shape=out_shape)(*args)


def kernel(x, params, edge_index):
    N, F = x.shape
    E = edge_index.shape[1]
    f32 = jnp.float32
    eidx = jnp.stack([edge_index[0].reshape(E // CH, CH),
                      edge_index[1].reshape(E // CH, CH)], axis=1)

    dims = [F] + [params[f"W{i}"].shape[1] for i in range(6)]
    dp = [max(d, 16) for d in dims]

    ws, bs, gs, betas = [], [], [], []
    for i in range(6):
        wi = params[f"W{i}"].astype(f32)
        ws.append(jnp.pad(wi, ((0, dp[i] - wi.shape[0]),
                               (0, dp[i + 1] - wi.shape[1]))))
        padc = dp[i + 1] - dims[i + 1]
        bs.append(jnp.pad(params[f"b{i}"], (0, padc))[None, :])
        gs.append(jnp.pad(params[f"g{i}"], (0, padc), constant_values=1.0)[None, :])
        betas.append(jnp.pad(params[f"beta{i}"], (0, padc))[None, :])
    a = params["a"].reshape(1, 1)
    wc = jnp.pad(params["Wc"], ((0, dp[6] - params["Wc"].shape[0]), (0, 0)))
    bc = params["bc"][None, :]

    ones16 = jnp.ones((CH, 16), f32)
    zeros = {d: jnp.zeros((64 if d > 64 else ZB, d), f32)
             for d in sorted(set(dp[1:]))}
    zeros16 = jnp.zeros((ZB, 16), f32)

    degs_s, degs_d = _sc_degrees(E, N)(eidx, ones16, zeros16)
    hpre, norms = _tc(
        functools.partial(_tc0_body, N),
        (jax.ShapeDtypeStruct((N, dp[1]), f32),
         jax.ShapeDtypeStruct((N, 2), f32)),
        x, degs_s, degs_d, ws[0])
    for i in range(6):
        d = dp[i + 1]
        parts = _sc_segsum(E, N, d)(hpre, eidx, zeros[d])
        if i < 5:
            hpre = _tc(
                functools.partial(_tc_mid_body, N),
                jax.ShapeDtypeStruct((N, dp[i + 2]), f32),
                parts, norms, bs[i], gs[i], betas[i], a, ws[i + 1])
        else:
            out = _tc(
                functools.partial(_tc_final_body, N),
                jax.ShapeDtypeStruct((1, bc.shape[1]), f32),
                parts, norms, bs[i], gs[i], betas[i], a, wc, bc)
    return out


# speedup vs baseline: 14.8305x; 1.0226x over previous
import functools

import jax
import jax.numpy as jnp
from jax import lax
from jax.experimental import pallas as pl
from jax.experimental.pallas import tpu as pltpu
from jax.experimental.pallas import tpu_sc as plsc

NC, NS = 2, 16
NW = NC * NS
CH = 128
ZB = 128
EPS = 1e-5


def _pad_n(n):
    q = NS * 8
    return (n + q - 1) // q * q


def _sc_mesh():
    return plsc.VectorSubcoreMesh(core_axis_name="c", subcore_axis_name="s")


@functools.lru_cache(maxsize=None)
def _sc_degrees(E, N):
    nchunk = E // CH
    K = 10
    nblk = nchunk // K
    assert nchunk % K == 0
    NP = _pad_n(N)
    rows_per = NP // NS

    @functools.partial(
        pl.kernel,
        out_type=(jax.ShapeDtypeStruct((NC, NP, 16), jnp.float32),
                  jax.ShapeDtypeStruct((NC, NP, 16), jnp.float32)),
        mesh=_sc_mesh(),
        scratch_types=(
            pltpu.VMEM((K, 2, CH), jnp.int32),
            pltpu.VMEM((CH, 16), jnp.float32),
            pltpu.VMEM_SHARED((NP, 16), jnp.float32),
            pltpu.VMEM_SHARED((NP, 16), jnp.float32),
            pltpu.SemaphoreType.DMA,
        ),
        compiler_params=pltpu.CompilerParams(use_tc_tiling_on_sc=False),
    )
    def deg_kernel(eidx_hbm, ones_hbm, zeros_hbm, outs, outd,
                   idxb, ones_v, acc_s, acc_d, sem):
        c = lax.axis_index("c")
        s = lax.axis_index("s")
        w = s * NC + c
        base = s * rows_per
        pltpu.sync_copy(zeros_hbm, acc_s.at[pl.ds(base, rows_per)])
        pltpu.sync_copy(zeros_hbm, acc_d.at[pl.ds(base, rows_per)])
        pltpu.sync_copy(ones_hbm, ones_v)
        plsc.subcore_barrier()

        cnt = (nblk - w + NW - 1) // NW

        def body(g, carry):
            blk = w + g * NW
            pltpu.sync_copy(eidx_hbm.at[pl.ds(blk * K, K)], idxb)
            hs = []
            for j in range(K):
                hs.append(pltpu.async_copy(
                    ones_v, acc_s.at[idxb.at[j, 0]], sem, add=True))
                hs.append(pltpu.async_copy(
                    ones_v, acc_d.at[idxb.at[j, 1]], sem, add=True))
            for h in hs:
                h.wait()
            return carry

        lax.fori_loop(0, cnt, body, 0)
        plsc.subcore_barrier()
        pltpu.sync_copy(acc_s.at[pl.ds(base, rows_per)],
                        outs.at[c, pl.ds(base, rows_per)])
        pltpu.sync_copy(acc_d.at[pl.ds(base, rows_per)],
                        outd.at[c, pl.ds(base, rows_per)])

    return deg_kernel


@functools.lru_cache(maxsize=None)
def _sc_segsum(E, N, D):
    nchunk = E // CH
    K = 3 if D > 64 else (5 if D > 32 else 10)
    nblk = nchunk // K
    rem = nchunk - nblk * K
    NP = _pad_n(N)
    rows_per = NP // NS

    @functools.partial(
        pl.kernel,
        out_type=jax.ShapeDtypeStruct((NC, NP, D), jnp.float32),
        mesh=_sc_mesh(),
        scratch_types=(
            pltpu.VMEM((K, 2, CH), jnp.int32),
            pltpu.VMEM((K, CH, D), jnp.float32),
            pltpu.VMEM_SHARED((NP, D), jnp.float32),
            pltpu.SemaphoreType.DMA,
            pltpu.SemaphoreType.DMA,
        ),
        compiler_params=pltpu.CompilerParams(use_tc_tiling_on_sc=False),
    )
    def seg_kernel(h_hbm, eidx_hbm, zeros_hbm, out,
                   idxb, rows, acc, gsem, ssem):
        c = lax.axis_index("c")
        s = lax.axis_index("s")
        w = s * NC + c
        base = s * rows_per
        pltpu.sync_copy(zeros_hbm, acc.at[pl.ds(base, rows_per)])
        plsc.subcore_barrier()

        cnt = (nblk - w + NW - 1) // NW

        def body(g, carry):
            blk = w + g * NW
            pltpu.sync_copy(eidx_hbm.at[pl.ds(blk * K, K)], idxb)
            gh = [pltpu.async_copy(h_hbm.at[idxb.at[j, 0]], rows.at[j], gsem)
                  for j in range(K)]
            sh = []
            for j in range(K):
                gh[j].wait()
                sh.append(pltpu.async_copy(
                    rows.at[j], acc.at[idxb.at[j, 1]], ssem, add=True))
            for h in sh:
                h.wait()
            return carry

        lax.fori_loop(0, cnt, body, 0)
        if rem:
            @pl.when(w < rem)
            def _():
                blk1 = nblk * K + w
                pltpu.sync_copy(eidx_hbm.at[pl.ds(blk1, 1)],
                                idxb.at[pl.ds(0, 1)])
                pltpu.async_copy(h_hbm.at[idxb.at[0, 0]], rows.at[0],
                                 gsem).wait()
                pltpu.sync_copy(rows.at[0], acc.at[idxb.at[0, 1]], add=True)
        plsc.subcore_barrier()
        pltpu.sync_copy(acc.at[pl.ds(base, rows_per)],
                        out.at[c, pl.ds(base, rows_per)])

    return seg_kernel


def _tc0_body(n, x_ref, ds_ref, dd_ref, w_ref, hpre_ref, norms_ref):
    d_s = ds_ref[...]
    d_d = dd_ref[...]
    deg_o = d_s[0, :n, 0:1] + d_s[1, :n, 0:1]
    deg_i = d_d[0, :n, 0:1] + d_d[1, :n, 0:1]
    ns = jnp.where(deg_o > 0, lax.rsqrt(jnp.maximum(deg_o, 1.0)), 0.0)
    nd = jnp.where(deg_i > 0, lax.rsqrt(jnp.maximum(deg_i, 1.0)), 0.0)
    norms_ref[...] = jnp.concatenate([ns, nd], axis=1)
    hpre_ref[...] = jnp.dot(x_ref[...] * ns, w_ref[...],
                            preferred_element_type=jnp.float32)


def _post_agg(n, parts_ref, norms_ref, b_ref, g_ref, beta_ref, a_ref):
    p = parts_ref[...]
    agg = p[0, :n] + p[1, :n]
    nrm = norms_ref[...]
    ns = nrm[:, 0:1]
    nd = nrm[:, 1:2]
    h = agg * nd + b_ref[...]
    mu = jnp.mean(h, axis=0, keepdims=True)
    var = jnp.mean((h - mu) ** 2, axis=0, keepdims=True)
    hn = (h - mu) * lax.rsqrt(var + EPS) * g_ref[...] + beta_ref[...]
    hp = jnp.where(hn >= 0, hn, a_ref[...] * hn)
    return hp, ns


def _tc_mid_body(n, parts_ref, norms_ref, b_ref, g_ref, beta_ref, a_ref,
                 w_ref, out_ref):
    hp, ns = _post_agg(n, parts_ref, norms_ref, b_ref, g_ref, beta_ref, a_ref)
    out_ref[...] = jnp.dot(hp * ns, w_ref[...],
                           preferred_element_type=jnp.float32)


def _tc_final_body(n, parts_ref, norms_ref, b_ref, g_ref, beta_ref, a_ref,
                   wc_ref, bc_ref, out_ref):
    hp, _ = _post_agg(n, parts_ref, norms_ref, b_ref, g_ref, beta_ref, a_ref)
    hg = jnp.mean(hp, axis=0, keepdims=True)
    out_ref[...] = jnp.dot(hg, wc_ref[...],
                           preferred_element_type=jnp.float32) + bc_ref[...]


def _tc(body, out_shape, *args):
    return pl.pallas_call(body, out_shape=out_shape)(*args)


def kernel(x, params, edge_index):
    N, F = x.shape
    E = edge_index.shape[1]
    f32 = jnp.float32
    eidx = jnp.stack([edge_index[0].reshape(E // CH, CH),
                      edge_index[1].reshape(E // CH, CH)], axis=1)

    dims = [F] + [params[f"W{i}"].shape[1] for i in range(6)]
    dp = [max(d, 16) for d in dims]

    ws, bs, gs, betas = [], [], [], []
    for i in range(6):
        wi = params[f"W{i}"].astype(f32)
        ws.append(jnp.pad(wi, ((0, dp[i] - wi.shape[0]),
                               (0, dp[i + 1] - wi.shape[1]))))
        padc = dp[i + 1] - dims[i + 1]
        bs.append(jnp.pad(params[f"b{i}"], (0, padc))[None, :])
        gs.append(jnp.pad(params[f"g{i}"], (0, padc), constant_values=1.0)[None, :])
        betas.append(jnp.pad(params[f"beta{i}"], (0, padc))[None, :])
    a = params["a"].reshape(1, 1)
    wc = jnp.pad(params["Wc"], ((0, dp[6] - params["Wc"].shape[0]), (0, 0)))
    bc = params["bc"][None, :]

    ones16 = jnp.ones((CH, 16), f32)
    rows_per = _pad_n(N) // NS
    zeros = {d: jnp.zeros((rows_per, d), f32) for d in sorted(set(dp[1:]))}
    zeros16 = jnp.zeros((rows_per, 16), f32)

    degs_s, degs_d = _sc_degrees(E, N)(eidx, ones16, zeros16)
    hpre, norms = _tc(
        functools.partial(_tc0_body, N),
        (jax.ShapeDtypeStruct((N, dp[1]), f32),
         jax.ShapeDtypeStruct((N, 2), f32)),
        x, degs_s, degs_d, ws[0])
    for i in range(6):
        d = dp[i + 1]
        parts = _sc_segsum(E, N, d)(hpre, eidx, zeros[d])
        if i < 5:
            hpre = _tc(
                functools.partial(_tc_mid_body, N),
                jax.ShapeDtypeStruct((N, dp[i + 2]), f32),
                parts, norms, bs[i], gs[i], betas[i], a, ws[i + 1])
        else:
            out = _tc(
                functools.partial(_tc_final_body, N),
                jax.ShapeDtypeStruct((1, bc.shape[1]), f32),
                parts, norms, bs[i], gs[i], betas[i], a, wc, bc)
    return out


# speedup vs baseline: 14.9716x; 1.0095x over previous
import functools

import jax
import jax.numpy as jnp
from jax import lax
from jax.experimental import pallas as pl
from jax.experimental.pallas import tpu as pltpu
from jax.experimental.pallas import tpu_sc as plsc

NC, NS = 2, 16
NW = NC * NS
CH = 128
ZB = 128
EPS = 1e-5


def _pad_n(n):
    q = NS * 8
    return (n + q - 1) // q * q


def _sc_mesh():
    return plsc.VectorSubcoreMesh(core_axis_name="c", subcore_axis_name="s")


@functools.lru_cache(maxsize=None)
def _sc_degrees(E, N):
    nchunk = E // CH
    K = 10
    nblk = nchunk // K
    assert nchunk % K == 0
    NP = _pad_n(N)
    rows_per = NP // NS

    @functools.partial(
        pl.kernel,
        out_type=(jax.ShapeDtypeStruct((NC, NP, 16), jnp.float32),
                  jax.ShapeDtypeStruct((NC, NP, 16), jnp.float32)),
        mesh=_sc_mesh(),
        scratch_types=(
            pltpu.VMEM((K, 2, CH), jnp.int32),
            pltpu.VMEM((CH, 16), jnp.float32),
            pltpu.VMEM_SHARED((NP, 16), jnp.float32),
            pltpu.VMEM_SHARED((NP, 16), jnp.float32),
            pltpu.SemaphoreType.DMA,
        ),
        compiler_params=pltpu.CompilerParams(use_tc_tiling_on_sc=False),
    )
    def deg_kernel(eidx_hbm, ones_hbm, zeros_hbm, outs, outd,
                   idxb, ones_v, acc_s, acc_d, sem):
        c = lax.axis_index("c")
        s = lax.axis_index("s")
        w = s * NC + c
        base = s * rows_per
        pltpu.sync_copy(zeros_hbm, acc_s.at[pl.ds(base, rows_per)])
        pltpu.sync_copy(zeros_hbm, acc_d.at[pl.ds(base, rows_per)])
        pltpu.sync_copy(ones_hbm, ones_v)
        plsc.subcore_barrier()

        cnt = (nblk - w + NW - 1) // NW

        def body(g, carry):
            blk = w + g * NW
            pltpu.sync_copy(eidx_hbm.at[pl.ds(blk * K, K)], idxb)
            hs = []
            for j in range(K):
                hs.append(pltpu.async_copy(
                    ones_v, acc_s.at[idxb.at[j, 0]], sem, add=True))
                hs.append(pltpu.async_copy(
                    ones_v, acc_d.at[idxb.at[j, 1]], sem, add=True))
            for h in hs:
                h.wait()
            return carry

        lax.fori_loop(0, cnt, body, 0)
        plsc.subcore_barrier()
        pltpu.sync_copy(acc_s.at[pl.ds(base, rows_per)],
                        outs.at[c, pl.ds(base, rows_per)])
        pltpu.sync_copy(acc_d.at[pl.ds(base, rows_per)],
                        outd.at[c, pl.ds(base, rows_per)])

    return deg_kernel


@functools.lru_cache(maxsize=None)
def _sc_segsum(E, N, D):
    nchunk = E // CH
    K = 3 if D > 64 else 10
    nblk = nchunk // K
    rem = nchunk - nblk * K
    NP = _pad_n(N)
    rows_per = NP // NS

    @functools.partial(
        pl.kernel,
        out_type=jax.ShapeDtypeStruct((NC, NP, D), jnp.float32),
        mesh=_sc_mesh(),
        scratch_types=(
            pltpu.VMEM((K, 2, CH), jnp.int32),
            pltpu.VMEM((K, CH, D), jnp.float32),
            pltpu.VMEM_SHARED((NP, D), jnp.float32),
            pltpu.SemaphoreType.DMA,
            pltpu.SemaphoreType.DMA,
        ),
        compiler_params=pltpu.CompilerParams(use_tc_tiling_on_sc=False),
    )
    def seg_kernel(h_hbm, eidx_hbm, zeros_hbm, out,
                   idxb, rows, acc, gsem, ssem):
        c = lax.axis_index("c")
        s = lax.axis_index("s")
        w = s * NC + c
        base = s * rows_per
        pltpu.sync_copy(zeros_hbm, acc.at[pl.ds(base, rows_per)])
        plsc.subcore_barrier()

        cnt = (nblk - w + NW - 1) // NW

        def body(g, carry):
            blk = w + g * NW
            pltpu.sync_copy(eidx_hbm.at[pl.ds(blk * K, K)], idxb)
            gh = [pltpu.async_copy(h_hbm.at[idxb.at[j, 0]], rows.at[j], gsem)
                  for j in range(K)]
            sh = []
            for j in range(K):
                gh[j].wait()
                sh.append(pltpu.async_copy(
                    rows.at[j], acc.at[idxb.at[j, 1]], ssem, add=True))
            for h in sh:
                h.wait()
            return carry

        lax.fori_loop(0, cnt, body, 0)
        if rem:
            @pl.when(w < rem)
            def _():
                blk1 = nblk * K + w
                pltpu.sync_copy(eidx_hbm.at[pl.ds(blk1, 1)],
                                idxb.at[pl.ds(0, 1)])
                pltpu.async_copy(h_hbm.at[idxb.at[0, 0]], rows.at[0],
                                 gsem).wait()
                pltpu.sync_copy(rows.at[0], acc.at[idxb.at[0, 1]], add=True)
        plsc.subcore_barrier()
        pltpu.sync_copy(acc.at[pl.ds(base, rows_per)],
                        out.at[c, pl.ds(base, rows_per)])

    return seg_kernel


def _tc0_body(n, x_ref, ds_ref, dd_ref, w_ref, hpre_ref, norms_ref):
    d_s = ds_ref[...]
    d_d = dd_ref[...]
    deg_o = d_s[0, :n, 0:1] + d_s[1, :n, 0:1]
    deg_i = d_d[0, :n, 0:1] + d_d[1, :n, 0:1]
    ns = jnp.where(deg_o > 0, lax.rsqrt(jnp.maximum(deg_o, 1.0)), 0.0)
    nd = jnp.where(deg_i > 0, lax.rsqrt(jnp.maximum(deg_i, 1.0)), 0.0)
    norms_ref[...] = jnp.concatenate([ns, nd], axis=1)
    hpre_ref[...] = jnp.dot(x_ref[...] * ns, w_ref[...],
                            preferred_element_type=jnp.float32)


def _post_agg(n, parts_ref, norms_ref, b_ref, g_ref, beta_ref, a_ref):
    p = parts_ref[...]
    agg = p[0, :n] + p[1, :n]
    nrm = norms_ref[...]
    ns = nrm[:, 0:1]
    nd = nrm[:, 1:2]
    h = agg * nd + b_ref[...]
    mu = jnp.mean(h, axis=0, keepdims=True)
    var = jnp.mean((h - mu) ** 2, axis=0, keepdims=True)
    hn = (h - mu) * lax.rsqrt(var + EPS) * g_ref[...] + beta_ref[...]
    hp = jnp.where(hn >= 0, hn, a_ref[...] * hn)
    return hp, ns


def _tc_mid_body(n, parts_ref, norms_ref, b_ref, g_ref, beta_ref, a_ref,
                 w_ref, out_ref):
    hp, ns = _post_agg(n, parts_ref, norms_ref, b_ref, g_ref, beta_ref, a_ref)
    out_ref[...] = jnp.dot(hp * ns, w_ref[...],
                           preferred_element_type=jnp.float32)


def _tc_final_body(n, parts_ref, norms_ref, b_ref, g_ref, beta_ref, a_ref,
                   wc_ref, bc_ref, out_ref):
    hp, _ = _post_agg(n, parts_ref, norms_ref, b_ref, g_ref, beta_ref, a_ref)
    hg = jnp.mean(hp, axis=0, keepdims=True)
    out_ref[...] = jnp.dot(hg, wc_ref[...],
                           preferred_element_type=jnp.float32) + bc_ref[...]


def _tc(body, out_shape, *args):
    return pl.pallas_call(body, out_shape=out_shape)(*args)


def kernel(x, params, edge_index):
    N, F = x.shape
    E = edge_index.shape[1]
    f32 = jnp.float32
    eidx = jnp.stack([edge_index[0].reshape(E // CH, CH),
                      edge_index[1].reshape(E // CH, CH)], axis=1)

    dims = [F] + [params[f"W{i}"].shape[1] for i in range(6)]
    dp = [max(d, 16) for d in dims]

    ws, bs, gs, betas = [], [], [], []
    for i in range(6):
        wi = params[f"W{i}"].astype(f32)
        ws.append(jnp.pad(wi, ((0, dp[i] - wi.shape[0]),
                               (0, dp[i + 1] - wi.shape[1]))))
        padc = dp[i + 1] - dims[i + 1]
        bs.append(jnp.pad(params[f"b{i}"], (0, padc))[None, :])
        gs.append(jnp.pad(params[f"g{i}"], (0, padc), constant_values=1.0)[None, :])
        betas.append(jnp.pad(params[f"beta{i}"], (0, padc))[None, :])
    a = params["a"].reshape(1, 1)
    wc = jnp.pad(params["Wc"], ((0, dp[6] - params["Wc"].shape[0]), (0, 0)))
    bc = params["bc"][None, :]

    ones16 = jnp.ones((CH, 16), f32)
    rows_per = _pad_n(N) // NS
    zeros = {d: jnp.zeros((rows_per, d), f32) for d in sorted(set(dp[1:]))}
    zeros16 = jnp.zeros((rows_per, 16), f32)

    degs_s, degs_d = _sc_degrees(E, N)(eidx, ones16, zeros16)
    hpre, norms = _tc(
        functools.partial(_tc0_body, N),
        (jax.ShapeDtypeStruct((N, dp[1]), f32),
         jax.ShapeDtypeStruct((N, 2), f32)),
        x, degs_s, degs_d, ws[0])
    for i in range(6):
        d = dp[i + 1]
        parts = _sc_segsum(E, N, d)(hpre, eidx, zeros[d])
        if i < 5:
            hpre = _tc(
                functools.partial(_tc_mid_body, N),
                jax.ShapeDtypeStruct((N, dp[i + 2]), f32),
                parts, norms, bs[i], gs[i], betas[i], a, ws[i + 1])
        else:
            out = _tc(
                functools.partial(_tc_final_body, N),
                jax.ShapeDtypeStruct((1, bc.shape[1]), f32),
                parts, norms, bs[i], gs[i], betas[i], a, wc, bc)
    return out
